# baseline (speedup 1.0000x reference)
import functools

import jax
import jax.numpy as jnp
from jax import lax
from jax.experimental import pallas as pl
from jax.experimental.pallas import tpu as pltpu
from jax.experimental.pallas import tpu_sc as plsc


def _make_sc_kernel(B, C, H, W, half):
    HW = H * W
    info = plsc.get_sparse_core_info()
    NC, NS, L = info.num_cores, info.num_subcores, info.num_lanes
    NW = NC * NS
    CPW = C // NW
    n_half_workers = half // CPW
    mesh = plsc.VectorSubcoreMesh(core_axis_name="c", subcore_axis_name="s")

    @functools.partial(
        pl.kernel,
        out_type=jax.ShapeDtypeStruct((B, C * HW), jnp.float32),
        mesh=mesh,
        compiler_params=pltpu.CompilerParams(needs_layout_passes=False),
        scratch_types=[
            pltpu.VMEM((H * half,), jnp.float32),
            pltpu.VMEM((CPW * HW,), jnp.float32),
            pltpu.SemaphoreType.DMA,
        ],
    )
    def sc_pos(row_hbm, col_hbm, out_hbm, tbl_v, chunk_v, sem):
        wid = lax.axis_index("s") * NC + lax.axis_index("c")
        is_row = wid < n_half_workers
        c0 = wid * CPW
        t0 = jnp.where(is_row, c0, c0 - half)

        @pl.when(is_row)
        def _():
            pltpu.sync_copy(row_hbm, tbl_v)

        @pl.when(jnp.logical_not(is_row))
        def _():
            pltpu.sync_copy(col_hbm, tbl_v)

        lanes = lax.broadcasted_iota(jnp.int32, (L,), 0)

        @pl.when(is_row)
        def _():
            def jbody(j, _):
                col_idx = jnp.full((L,), t0 + j, dtype=jnp.int32)
                for h in range(H):
                    vec = plsc.load_gather(tbl_v, [col_idx + h * half])
                    for q in range(W // L):
                        chunk_v[pl.ds(j * HW + h * W + q * L, L)] = vec
                return 0
            lax.fori_loop(0, CPW, jbody, 0)

        @pl.when(jnp.logical_not(is_row))
        def _():
            def jbody(j, _):
                col_idx = jnp.full((L,), t0 + j, dtype=jnp.int32)
                vecs = [plsc.load_gather(tbl_v, [(lanes + q * L) * half + col_idx])
                        for q in range(W // L)]
                for h in range(H):
                    for q in range(W // L):
                        chunk_v[pl.ds(j * HW + h * W + q * L, L)] = vecs[q]
                return 0
            lax.fori_loop(0, CPW, jbody, 0)

        copies = [
            pltpu.make_async_copy(
                chunk_v, out_hbm.at[b].at[pl.ds(c0 * HW, CPW * HW)], sem)
            for b in range(B)
        ]
        for c in copies:
            c.start()
        for c in copies:
            c.wait()

    return sc_pos


def kernel(feat, row_embed, col_embed):
    B, C, H, W = feat.shape
    half = row_embed.shape[1]
    sc_pos = _make_sc_kernel(B, C, H, W, half)
    out = sc_pos(row_embed[:H].reshape(-1), col_embed[:W].reshape(-1))
    return out.reshape(B, C, H, W)


# speedup vs baseline: 1.3731x; 1.3731x over previous
import functools

import jax
import jax.numpy as jnp
from jax import lax
from jax.experimental import pallas as pl
from jax.experimental.pallas import tpu as pltpu
from jax.experimental.pallas import tpu_sc as plsc


def _make_sc_kernel(B, C, H, W, half):
    HW = H * W
    info = plsc.get_sparse_core_info()
    NC, NS, L = info.num_cores, info.num_subcores, info.num_lanes
    NW = NC * NS
    CPW = C // NW
    n_half_workers = half // CPW
    mesh = plsc.VectorSubcoreMesh(core_axis_name="c", subcore_axis_name="s")

    @functools.partial(
        pl.kernel,
        out_type=jax.ShapeDtypeStruct((B * C * HW,), jnp.float32),
        mesh=mesh,
        compiler_params=pltpu.CompilerParams(needs_layout_passes=False),
        scratch_types=[
            pltpu.VMEM((H * half,), jnp.float32),
            pltpu.VMEM((CPW * HW,), jnp.float32),
            pltpu.SemaphoreType.DMA,
        ],
    )
    def sc_pos(row_hbm, col_hbm, out_hbm, tbl_v, chunk_v, sem):
        wid = lax.axis_index("s") * NC + lax.axis_index("c")
        is_row = wid < n_half_workers
        c0 = wid * CPW
        t0 = jnp.where(is_row, c0, c0 - half)

        @pl.when(is_row)
        def _():
            pltpu.sync_copy(row_hbm, tbl_v)

        @pl.when(jnp.logical_not(is_row))
        def _():
            pltpu.sync_copy(col_hbm, tbl_v)

        lanes = lax.broadcasted_iota(jnp.int32, (L,), 0)

        @pl.when(is_row)
        def _():
            def jbody(j, _):
                col_idx = jnp.full((L,), t0 + j, dtype=jnp.int32)
                for h in range(H):
                    vec = plsc.load_gather(tbl_v, [col_idx + h * half])
                    for q in range(W // L):
                        chunk_v[pl.ds(j * HW + h * W + q * L, L)] = vec
                return 0
            lax.fori_loop(0, CPW, jbody, 0)

        @pl.when(jnp.logical_not(is_row))
        def _():
            def jbody(j, _):
                col_idx = jnp.full((L,), t0 + j, dtype=jnp.int32)
                vecs = [plsc.load_gather(tbl_v, [(lanes + q * L) * half + col_idx])
                        for q in range(W // L)]
                for h in range(H):
                    for q in range(W // L):
                        chunk_v[pl.ds(j * HW + h * W + q * L, L)] = vecs[q]
                return 0
            lax.fori_loop(0, CPW, jbody, 0)

        copies = [
            pltpu.make_async_copy(
                chunk_v, out_hbm.at[pl.ds(b * C * HW + c0 * HW, CPW * HW)], sem)
            for b in range(B)
        ]
        for c in copies:
            c.start()
        for c in copies:
            c.wait()

    return sc_pos


def kernel(feat, row_embed, col_embed):
    B, C, H, W = feat.shape
    half = row_embed.shape[1]
    sc_pos = _make_sc_kernel(B, C, H, W, half)
    out = sc_pos(row_embed[:H].reshape(-1), col_embed[:W].reshape(-1))
    return out.reshape(B, C, H, W)


# speedup vs baseline: 6.6024x; 4.8085x over previous
import functools

import jax
import jax.numpy as jnp
from jax import lax
from jax.experimental import pallas as pl
from jax.experimental.pallas import tpu as pltpu
from jax.experimental.pallas import tpu_sc as plsc

_NSTRIP = 8


def _make_sc_kernel(B, C, H, W, half):
    info = plsc.get_sparse_core_info()
    NC, NS, L = info.num_cores, info.num_subcores, info.num_lanes
    NW = NC * NS
    HPW = (B * H) // NW
    mesh = plsc.VectorSubcoreMesh(core_axis_name="c", subcore_axis_name="s")

    @functools.partial(
        pl.kernel,
        out_type=jax.ShapeDtypeStruct((B, H, W, C), jnp.float32),
        mesh=mesh,
        compiler_params=pltpu.CompilerParams(needs_layout_passes=False),
        scratch_types=[
            pltpu.VMEM((H, half), jnp.float32),
            pltpu.VMEM((W, half), jnp.float32),
            pltpu.VMEM((_NSTRIP, W, half), jnp.float32),
            pltpu.SemaphoreType.DMA,
            pltpu.SemaphoreType.DMA((_NSTRIP,)),
        ],
    )
    def sc_pos(row_hbm, col_hbm, out_hbm, row_v, col_v, strips, csem, ssem):
        wid = lax.axis_index("s") * NC + lax.axis_index("c")
        b = wid // (H // HPW)
        h0 = (wid % (H // HPW)) * HPW
        pltpu.sync_copy(row_hbm, row_v)
        pltpu.sync_copy(col_hbm, col_v)

        col_copies = [
            pltpu.make_async_copy(
                col_v, out_hbm.at[b, h0 + i, :, pl.ds(half, half)], csem)
            for i in range(HPW)
        ]
        for c in col_copies:
            c.start()

        def ibody(i, _):
            s = lax.rem(i, _NSTRIP)

            @pl.when(i >= _NSTRIP)
            def _():
                pltpu.make_async_copy(
                    strips.at[s], out_hbm.at[b, h0, :, pl.ds(0, half)],
                    ssem.at[s]).wait()

            h = h0 + i
            for cc in range(half // L):
                vec = row_v[h, pl.ds(cc * L, L)]
                for w in range(W):
                    strips[s, w, pl.ds(cc * L, L)] = vec
            pltpu.make_async_copy(
                strips.at[s], out_hbm.at[b, h, :, pl.ds(0, half)],
                ssem.at[s]).start()
            return 0

        lax.fori_loop(0, HPW, ibody, 0, unroll=False)

        for s in range(_NSTRIP):
            pltpu.make_async_copy(
                strips.at[s], out_hbm.at[b, h0, :, pl.ds(0, half)],
                ssem.at[s]).wait()
        for c in col_copies:
            c.wait()

    return sc_pos


def kernel(feat, row_embed, col_embed):
    B, C, H, W = feat.shape
    half = row_embed.shape[1]
    sc_pos = _make_sc_kernel(B, C, H, W, half)
    out = sc_pos(row_embed[:H], col_embed[:W])
    return jnp.transpose(out, (0, 3, 1, 2))
